# baseline (speedup 1.0000x reference)
import functools

import jax
import jax.numpy as jnp
from jax import lax
from jax.experimental import pallas as pl
from jax.experimental.pallas import tpu as pltpu
from jax.experimental.pallas import tpu_sc as plsc

BATCH = 16384
EMBED_DIM = 64
L = 16
CH = 128


def kernel(user, item, user_table, item_table):
    info = plsc.get_sparse_core_info()
    nw = info.num_cores * info.num_subcores
    b_per_w = BATCH // nw
    nch = b_per_w // CH

    mesh = plsc.VectorSubcoreMesh(core_axis_name="c", subcore_axis_name="s")

    @functools.partial(
        pl.kernel,
        mesh=mesh,
        out_type=(
            jax.ShapeDtypeStruct((BATCH, EMBED_DIM), jnp.float32),
            jax.ShapeDtypeStruct((BATCH, EMBED_DIM), jnp.float32),
        ),
        scratch_types=[
            pltpu.VMEM((b_per_w,), jnp.int32),
            pltpu.VMEM((b_per_w,), jnp.int32),
            pltpu.VMEM((2, CH, EMBED_DIM), jnp.float32),
            pltpu.VMEM((2, CH, EMBED_DIM), jnp.float32),
            pltpu.SemaphoreType.DMA,
            pltpu.SemaphoreType.DMA,
            [pltpu.SemaphoreType.DMA] * 2,
            [pltpu.SemaphoreType.DMA] * 2,
        ],
    )
    def _lookup(user_hbm, item_hbm, ut_hbm, it_hbm, out_u, out_i,
                uidx_v, iidx_v, ubuf, ibuf, su, si, wu_sems, wi_sems):
        wid = lax.axis_index("s") * info.num_cores + lax.axis_index("c")
        base = wid * b_per_w
        pltpu.sync_copy(user_hbm.at[pl.ds(base, b_per_w)], uidx_v)
        pltpu.sync_copy(item_hbm.at[pl.ds(base, b_per_w)], iidx_v)

        wb_u = {}
        wb_i = {}
        for c in range(nch):
            h = c % 2
            if c >= 2:
                wb_u.pop(h).wait()
                wb_i.pop(h).wait()

            def grp(g, c=c, h=h):
                uv = uidx_v[pl.ds(c * CH + g * L, L)]
                iv = iidx_v[pl.ds(c * CH + g * L, L)]
                cps = []
                for k in range(L):
                    cps.append(pltpu.async_copy(
                        ut_hbm.at[pl.ds(uv[k], 1)],
                        ubuf.at[h].at[pl.ds(g * L + k, 1)], su))
                    cps.append(pltpu.async_copy(
                        it_hbm.at[pl.ds(iv[k], 1)],
                        ibuf.at[h].at[pl.ds(g * L + k, 1)], si))
                for cp in cps:
                    cp.wait()

            pl.loop(0, CH // L)(grp)
            obase = base + c * CH
            wb_u[h] = pltpu.async_copy(
                ubuf.at[h], out_u.at[pl.ds(obase, CH)], wu_sems[h])
            wb_i[h] = pltpu.async_copy(
                ibuf.at[h], out_i.at[pl.ds(obase, CH)], wi_sems[h])
        for cp in list(wb_u.values()) + list(wb_i.values()):
            cp.wait()

    return _lookup(user, item, user_table, item_table)
